# baseline (device time: 65051 ns/iter reference)
import jax
import jax.numpy as jnp
from jax import lax
from jax.experimental import pallas as pl
from jax.experimental.pallas import tpu as pltpu

N_DEV = 32

try:
    _ds = jax.devices()
    _coords = sorted({tuple(getattr(d, "coords", ())) for d in _ds})
    print(
        f"[kernel.py] n_devices={len(_ds)} kind={getattr(_ds[0], 'device_kind', '?')} "
        f"n_chips={len(_coords)} coords_span="
        f"{[tuple(max(c[i] for c in _coords) + 1 for i in range(3))] if _coords and len(_coords[0]) == 3 else _coords[:4]}"
    )
except Exception as _e:
    print(f"[kernel.py] topology probe failed: {_e!r}")


def kernel(x, dy):
    k, d = x.shape
    _, f = dy.shape
    m = d // N_DEV

    def body(x_ref, dy_ref, out_ref, acc_ref, comm_ref, send_sems, recv_sems):
        my = lax.axis_index("i")

        acc_ref[...] = lax.dot_general(
            x_ref[...].astype(jnp.bfloat16),
            dy_ref[...].astype(jnp.bfloat16),
            dimension_numbers=(((0,), (0,)), ((), ())),
            preferred_element_type=jnp.float32,
        )

        barrier = pltpu.get_barrier_semaphore()
        for o in range(1, N_DEV):
            pl.semaphore_signal(
                barrier,
                inc=1,
                device_id=((my + o) % N_DEV,),
                device_id_type=pl.DeviceIdType.MESH,
            )
        pl.semaphore_wait(barrier, N_DEV - 1)

        rdmas = []
        for o in range(1, N_DEV):
            dst = (my + o) % N_DEV
            rdma = pltpu.make_async_remote_copy(
                src_ref=acc_ref.at[pl.ds(dst * m, m), :],
                dst_ref=comm_ref.at[o - 1],
                send_sem=send_sems.at[o - 1],
                recv_sem=recv_sems.at[o - 1],
                device_id=(dst,),
                device_id_type=pl.DeviceIdType.MESH,
            )
            rdma.start()
            rdmas.append(rdma)

        out_ref[...] = acc_ref[pl.ds(my * m, m), :]
        for o in range(1, N_DEV):
            rdmas[o - 1].wait_recv()
            out_ref[...] += comm_ref[o - 1]
        for o in range(1, N_DEV):
            rdmas[o - 1].wait_send()

    return pl.pallas_call(
        body,
        out_shape=jax.ShapeDtypeStruct((m, f), jnp.float32),
        in_specs=[
            pl.BlockSpec(memory_space=pltpu.VMEM),
            pl.BlockSpec(memory_space=pltpu.VMEM),
        ],
        out_specs=pl.BlockSpec(memory_space=pltpu.VMEM),
        scratch_shapes=[
            pltpu.VMEM((d, f), jnp.float32),
            pltpu.VMEM((N_DEV - 1, m, f), jnp.float32),
            pltpu.SemaphoreType.DMA((N_DEV - 1,)),
            pltpu.SemaphoreType.DMA((N_DEV - 1,)),
        ],
        compiler_params=pltpu.CompilerParams(collective_id=0),
    )(x, dy)


# device time: 36500 ns/iter; 1.7822x vs baseline; 1.7822x over previous
import jax
import jax.numpy as jnp
from jax import lax
from jax.experimental import pallas as pl
from jax.experimental.pallas import tpu as pltpu

N_DEV = 32

try:
    _ds = jax.devices()
    _coords = sorted({tuple(getattr(d, "coords", ())) for d in _ds})
    print(
        f"[kernel.py] n_devices={len(_ds)} kind={getattr(_ds[0], 'device_kind', '?')} "
        f"n_chips={len(_coords)} coords_span="
        f"{[tuple(max(c[i] for c in _coords) + 1 for i in range(3))] if _coords and len(_coords[0]) == 3 else _coords[:4]}"
    )
except Exception as _e:
    print(f"[kernel.py] topology probe failed: {_e!r}")


def kernel(x, dy):
    k, d = x.shape
    _, f = dy.shape
    m = d // N_DEV

    def body(
        x_ref, dy_ref, out_ref, acc_ref, stage_ref, comm_ref, send_sems, recv_sems
    ):
        my = lax.axis_index("i")

        barrier = pltpu.get_barrier_semaphore()
        for o in range(1, N_DEV):
            pl.semaphore_signal(
                barrier,
                inc=1,
                device_id=((my + o) % N_DEV,),
                device_id_type=pl.DeviceIdType.MESH,
            )

        acc_ref[...] = lax.dot_general(
            x_ref[...].astype(jnp.bfloat16),
            dy_ref[...].astype(jnp.bfloat16),
            dimension_numbers=(((0,), (0,)), ((), ())),
            preferred_element_type=jnp.float32,
        )
        stage_ref[...] = acc_ref[...].astype(jnp.bfloat16)

        pl.semaphore_wait(barrier, N_DEV - 1)

        rdmas = []
        for o in range(1, N_DEV):
            dst = (my + o) % N_DEV
            rdma = pltpu.make_async_remote_copy(
                src_ref=stage_ref.at[pl.ds(dst * m, m), :],
                dst_ref=comm_ref.at[o - 1],
                send_sem=send_sems.at[o - 1],
                recv_sem=recv_sems.at[o - 1],
                device_id=(dst,),
                device_id_type=pl.DeviceIdType.MESH,
            )
            rdma.start()
            rdmas.append(rdma)

        out_ref[...] = acc_ref[pl.ds(my * m, m), :]
        for o in range(1, N_DEV):
            rdmas[o - 1].wait_recv()
            out_ref[...] += comm_ref[o - 1].astype(jnp.float32)
        for o in range(1, N_DEV):
            rdmas[o - 1].wait_send()

    return pl.pallas_call(
        body,
        out_shape=jax.ShapeDtypeStruct((m, f), jnp.float32),
        in_specs=[
            pl.BlockSpec(memory_space=pltpu.VMEM),
            pl.BlockSpec(memory_space=pltpu.VMEM),
        ],
        out_specs=pl.BlockSpec(memory_space=pltpu.VMEM),
        scratch_shapes=[
            pltpu.VMEM((d, f), jnp.float32),
            pltpu.VMEM((d, f), jnp.bfloat16),
            pltpu.VMEM((N_DEV - 1, m, f), jnp.bfloat16),
            pltpu.SemaphoreType.DMA((N_DEV - 1,)),
            pltpu.SemaphoreType.DMA((N_DEV - 1,)),
        ],
        compiler_params=pltpu.CompilerParams(collective_id=0),
    )(x, dy)


# device time: 28162 ns/iter; 2.3099x vs baseline; 1.2961x over previous
import jax
import jax.numpy as jnp
from jax import lax
from jax.experimental import pallas as pl
from jax.experimental.pallas import tpu as pltpu

N_DEV = 32
N_PLANE = 16


def _logical(x, y, z):
    return z * 8 + y * 2 + jnp.where(y % 2 == 0, x, 1 - x)


def kernel(x, dy):
    k, d = x.shape
    _, f = dy.shape
    m = d // N_DEV

    def body(
        x_ref,
        dy_ref,
        out_ref,
        acc_ref,
        stage_ref,
        piece_ref,
        fstage_ref,
        comm_ref,
        piece_send_sems,
        piece_recv_sems,
        fwd_send_sems,
        fwd_recv_sems,
    ):
        my = lax.axis_index("i")
        mz = my // 8
        rem = my % 8
        myy = rem // 2
        xp = rem % 2
        mx = jnp.where(myy % 2 == 0, xp, 1 - xp)
        p = mz * 4 + myy
        partner = _logical(1 - mx, myy, mz)

        barrier = pltpu.get_barrier_semaphore()
        for o in range(1, N_DEV):
            pl.semaphore_signal(
                barrier,
                inc=1,
                device_id=((my + o) % N_DEV,),
                device_id_type=pl.DeviceIdType.MESH,
            )

        acc_ref[...] = lax.dot_general(
            x_ref[...].astype(jnp.bfloat16),
            dy_ref[...].astype(jnp.bfloat16),
            dimension_numbers=(((0,), (0,)), ((), ())),
            preferred_element_type=jnp.float32,
        )
        stage_ref[...] = acc_ref[...].astype(jnp.bfloat16)

        pl.semaphore_wait(barrier, N_DEV - 1)

        def plane_member(q, layer_x):
            return _logical(layer_x, q % 4, q // 4)

        piece_rdmas = []
        for o in range(1, N_PLANE + 1):
            q = (p + o) % N_PLANE
            owner = plane_member(q, 1 - mx)
            rdma = pltpu.make_async_remote_copy(
                src_ref=stage_ref.at[pl.ds(owner * m, m), :],
                dst_ref=piece_ref.at[o - 1],
                send_sem=piece_send_sems.at[o - 1],
                recv_sem=piece_recv_sems.at[o - 1],
                device_id=(partner,),
                device_id_type=pl.DeviceIdType.MESH,
            )
            rdma.start()
            piece_rdmas.append(rdma)

        fwd_rdmas = []
        for o in range(1, N_PLANE):
            q = (p + o) % N_PLANE
            owner = plane_member(q, mx)
            piece_rdmas[o - 1].wait_recv()
            fstage_ref[o - 1] = (
                piece_ref[o - 1].astype(jnp.float32)
                + acc_ref[pl.ds(owner * m, m), :]
            ).astype(jnp.bfloat16)
            rdma = pltpu.make_async_remote_copy(
                src_ref=fstage_ref.at[o - 1],
                dst_ref=comm_ref.at[o - 1],
                send_sem=fwd_send_sems.at[o - 1],
                recv_sem=fwd_recv_sems.at[o - 1],
                device_id=(owner,),
                device_id_type=pl.DeviceIdType.MESH,
            )
            rdma.start()
            fwd_rdmas.append(rdma)

        piece_rdmas[N_PLANE - 1].wait_recv()
        out_ref[...] = (
            acc_ref[pl.ds(my * m, m), :]
            + piece_ref[N_PLANE - 1].astype(jnp.float32)
        )
        for o in range(1, N_PLANE):
            fwd_rdmas[o - 1].wait_recv()
            out_ref[...] += comm_ref[o - 1].astype(jnp.float32)

        for r in piece_rdmas:
            r.wait_send()
        for r in fwd_rdmas:
            r.wait_send()

    return pl.pallas_call(
        body,
        out_shape=jax.ShapeDtypeStruct((m, f), jnp.float32),
        in_specs=[
            pl.BlockSpec(memory_space=pltpu.VMEM),
            pl.BlockSpec(memory_space=pltpu.VMEM),
        ],
        out_specs=pl.BlockSpec(memory_space=pltpu.VMEM),
        scratch_shapes=[
            pltpu.VMEM((d, f), jnp.float32),
            pltpu.VMEM((d, f), jnp.bfloat16),
            pltpu.VMEM((N_PLANE, m, f), jnp.bfloat16),
            pltpu.VMEM((N_PLANE - 1, m, f), jnp.bfloat16),
            pltpu.VMEM((N_PLANE - 1, m, f), jnp.bfloat16),
            pltpu.SemaphoreType.DMA((N_PLANE,)),
            pltpu.SemaphoreType.DMA((N_PLANE,)),
            pltpu.SemaphoreType.DMA((N_PLANE - 1,)),
            pltpu.SemaphoreType.DMA((N_PLANE - 1,)),
        ],
        compiler_params=pltpu.CompilerParams(collective_id=0),
    )(x, dy)
